# Initial kernel scaffold; baseline (speedup 1.0000x reference)
import jax
import jax.numpy as jnp
from jax.experimental import pallas as pl


def kernel(x, W):
    raise NotImplementedError("write your pallas kernel here")


# speedup vs baseline: 1.1945x; 1.1945x over previous
import functools

import jax
import jax.numpy as jnp
from jax.experimental import pallas as pl
from jax.experimental.pallas import tpu as pltpu

HIDDEN = 4096
NUM_EXPERTS = 64
TOP_K = 8


def _gate_kernel(x_ref, w_ref, scores_ref, idx_ref):
    logits = jax.lax.dot_general(
        x_ref[...], w_ref[...],
        dimension_numbers=(((1,), (1,)), ((), ())),
        preferred_element_type=jnp.float32,
    )
    m = jnp.max(logits, axis=1, keepdims=True)
    e = jnp.exp(logits - m)
    p = e / jnp.sum(e, axis=1, keepdims=True)

    rows = p.shape[0]
    iota = jax.lax.broadcasted_iota(jnp.int32, (rows, NUM_EXPERTS), 1)
    work = p
    vals = []
    idxs = []
    for _ in range(TOP_K):
        mk = jnp.max(work, axis=1, keepdims=True)
        hit = work == mk
        ik = jnp.min(jnp.where(hit, iota, NUM_EXPERTS), axis=1, keepdims=True)
        vals.append(mk)
        idxs.append(ik)
        work = jnp.where(iota == ik, -1.0, work)

    top_p = jnp.concatenate(vals, axis=1)
    m2 = jnp.max(top_p, axis=1, keepdims=True)
    e2 = jnp.exp(top_p - m2)
    scores_ref[...] = e2 / jnp.sum(e2, axis=1, keepdims=True)
    idx_ref[...] = jnp.concatenate(idxs, axis=1)


@functools.partial(jax.jit, static_argnames=("block_rows",))
def _gate(x_flat, W, block_rows):
    rows = x_flat.shape[0]
    grid = (rows // block_rows,)
    return pl.pallas_call(
        _gate_kernel,
        grid=grid,
        in_specs=[
            pl.BlockSpec((block_rows, HIDDEN), lambda i: (i, 0)),
            pl.BlockSpec((NUM_EXPERTS, HIDDEN), lambda i: (0, 0)),
        ],
        out_specs=[
            pl.BlockSpec((block_rows, TOP_K), lambda i: (i, 0)),
            pl.BlockSpec((block_rows, TOP_K), lambda i: (i, 0)),
        ],
        out_shape=[
            jax.ShapeDtypeStruct((rows, TOP_K), jnp.float32),
            jax.ShapeDtypeStruct((rows, TOP_K), jnp.int32),
        ],
        compiler_params=pltpu.CompilerParams(
            dimension_semantics=("arbitrary",),
        ),
    )(x_flat, W)


def kernel(x, W):
    batch, seq, hidden = x.shape
    x_flat = x.reshape(batch * seq, hidden)
    top_scores, top_idx = _gate(x_flat, W, 512)
    aux = jnp.zeros((), dtype=x.dtype)
    return (top_scores, top_idx, aux)


# speedup vs baseline: 1.1955x; 1.0008x over previous
import functools

import jax
import jax.numpy as jnp
from jax.experimental import pallas as pl
from jax.experimental.pallas import tpu as pltpu

HIDDEN = 4096
NUM_EXPERTS = 64
TOP_K = 8


def _gate_kernel(x_ref, w_ref, scores_ref, idx_ref):
    logits = jax.lax.dot_general(
        x_ref[...], w_ref[...],
        dimension_numbers=(((1,), (1,)), ((), ())),
        preferred_element_type=jnp.float32,
    )
    m = jnp.max(logits, axis=1, keepdims=True)
    e = jnp.exp(logits - m)
    p = e / jnp.sum(e, axis=1, keepdims=True)

    rows = p.shape[0]
    iota = jax.lax.broadcasted_iota(jnp.int32, (rows, NUM_EXPERTS), 1)
    work = p
    vals = []
    idxs = []
    for _ in range(TOP_K):
        mk = jnp.max(work, axis=1, keepdims=True)
        hit = work == mk
        ik = jnp.min(jnp.where(hit, iota, NUM_EXPERTS), axis=1, keepdims=True)
        vals.append(mk)
        idxs.append(ik)
        work = jnp.where(iota == ik, -1.0, work)

    top_p = jnp.concatenate(vals, axis=1)
    m2 = jnp.max(top_p, axis=1, keepdims=True)
    e2 = jnp.exp(top_p - m2)
    scores_ref[...] = e2 / jnp.sum(e2, axis=1, keepdims=True)
    idx_ref[...] = jnp.concatenate(idxs, axis=1)


@functools.partial(jax.jit, static_argnames=("block_rows",))
def _gate(x_flat, W, block_rows):
    rows = x_flat.shape[0]
    grid = (rows // block_rows,)
    return pl.pallas_call(
        _gate_kernel,
        grid=grid,
        in_specs=[
            pl.BlockSpec((block_rows, HIDDEN), lambda i: (i, 0)),
            pl.BlockSpec((NUM_EXPERTS, HIDDEN), lambda i: (0, 0)),
        ],
        out_specs=[
            pl.BlockSpec((block_rows, TOP_K), lambda i: (i, 0)),
            pl.BlockSpec((block_rows, TOP_K), lambda i: (i, 0)),
        ],
        out_shape=[
            jax.ShapeDtypeStruct((rows, TOP_K), jnp.float32),
            jax.ShapeDtypeStruct((rows, TOP_K), jnp.int32),
        ],
        compiler_params=pltpu.CompilerParams(
            dimension_semantics=("parallel",),
        ),
    )(x_flat, W)


def kernel(x, W):
    batch, seq, hidden = x.shape
    x_flat = x.reshape(batch * seq, hidden)
    top_scores, top_idx = _gate(x_flat, W, 512)
    aux = jnp.zeros((), dtype=x.dtype)
    return (top_scores, top_idx, aux)


# speedup vs baseline: 1.4923x; 1.2483x over previous
import functools

import jax
import jax.numpy as jnp
from jax.experimental import pallas as pl
from jax.experimental.pallas import tpu as pltpu

HIDDEN = 4096
NUM_EXPERTS = 64
TOP_K = 8


def _gate_kernel(x_ref, w_ref, scores_ref, idx_ref):
    logits = jax.lax.dot_general(
        w_ref[...], x_ref[...],
        dimension_numbers=(((1,), (1,)), ((), ())),
        preferred_element_type=jnp.float32,
    )
    m = jnp.max(logits, axis=0, keepdims=True)
    e = jnp.exp(logits - m)
    p = e / jnp.sum(e, axis=0, keepdims=True)

    cols = p.shape[1]
    iota = jax.lax.broadcasted_iota(jnp.int32, (NUM_EXPERTS, cols), 0)
    work = p
    vals = []
    idxs = []
    for _ in range(TOP_K):
        mk = jnp.max(work, axis=0, keepdims=True)
        hit = work == mk
        ik = jnp.min(jnp.where(hit, iota, NUM_EXPERTS), axis=0, keepdims=True)
        vals.append(mk)
        idxs.append(ik)
        work = jnp.where(iota == ik, -1.0, work)

    top_p = jnp.concatenate(vals, axis=0)
    m2 = jnp.max(top_p, axis=0, keepdims=True)
    e2 = jnp.exp(top_p - m2)
    s = e2 / jnp.sum(e2, axis=0, keepdims=True)
    scores_ref[...] = s.T
    idx_ref[...] = jnp.concatenate(idxs, axis=0).T


@functools.partial(jax.jit, static_argnames=("block_rows",))
def _gate(x_flat, W, block_rows):
    rows = x_flat.shape[0]
    grid = (rows // block_rows,)
    return pl.pallas_call(
        _gate_kernel,
        grid=grid,
        in_specs=[
            pl.BlockSpec((block_rows, HIDDEN), lambda i: (i, 0)),
            pl.BlockSpec((NUM_EXPERTS, HIDDEN), lambda i: (0, 0)),
        ],
        out_specs=[
            pl.BlockSpec((block_rows, TOP_K), lambda i: (i, 0)),
            pl.BlockSpec((block_rows, TOP_K), lambda i: (i, 0)),
        ],
        out_shape=[
            jax.ShapeDtypeStruct((rows, TOP_K), jnp.float32),
            jax.ShapeDtypeStruct((rows, TOP_K), jnp.int32),
        ],
        compiler_params=pltpu.CompilerParams(
            dimension_semantics=("parallel",),
        ),
    )(x_flat, W)


def kernel(x, W):
    batch, seq, hidden = x.shape
    x_flat = x.reshape(batch * seq, hidden)
    top_scores, top_idx = _gate(x_flat, W, 512)
    aux = jnp.zeros((), dtype=x.dtype)
    return (top_scores, top_idx, aux)


# speedup vs baseline: 1.5136x; 1.0142x over previous
import functools

import jax
import jax.numpy as jnp
from jax.experimental import pallas as pl
from jax.experimental.pallas import tpu as pltpu

HIDDEN = 4096
NUM_EXPERTS = 64
TOP_K = 8


N_CHUNKS = 4


def _gate_kernel(x_ref, w_ref, scores_ref, idx_ref):
    rows = x_ref.shape[0]
    c = rows // N_CHUNKS
    w = w_ref[...]
    iota = jax.lax.broadcasted_iota(jnp.int32, (NUM_EXPERTS, c), 0)
    logits = []
    for j in range(N_CHUNKS):
        logits.append(jax.lax.dot_general(
            w, x_ref[j * c:(j + 1) * c, :],
            dimension_numbers=(((1,), (1,)), ((), ())),
            preferred_element_type=jnp.float32,
        ))
    for j in range(N_CHUNKS):
        l = logits[j]
        m = jnp.max(l, axis=0, keepdims=True)
        e = jnp.exp(l - m)
        p = e / jnp.sum(e, axis=0, keepdims=True)
        work = p
        vals = []
        idxs = []
        for _ in range(TOP_K):
            mk = jnp.max(work, axis=0, keepdims=True)
            hit = work == mk
            ik = jnp.min(jnp.where(hit, iota, NUM_EXPERTS), axis=0,
                         keepdims=True)
            vals.append(mk)
            idxs.append(ik)
            work = jnp.where(iota == ik, -1.0, work)
        top_p = jnp.concatenate(vals, axis=0)
        m2 = jnp.max(top_p, axis=0, keepdims=True)
        e2 = jnp.exp(top_p - m2)
        s = e2 / jnp.sum(e2, axis=0, keepdims=True)
        scores_ref[j * c:(j + 1) * c, :] = s.T
        idx_ref[j * c:(j + 1) * c, :] = jnp.concatenate(idxs, axis=0).T


@functools.partial(jax.jit, static_argnames=("block_rows",))
def _gate(x_flat, W, block_rows):
    rows = x_flat.shape[0]
    grid = (rows // block_rows,)
    return pl.pallas_call(
        _gate_kernel,
        grid=grid,
        in_specs=[
            pl.BlockSpec((block_rows, HIDDEN), lambda i: (i, 0)),
            pl.BlockSpec((NUM_EXPERTS, HIDDEN), lambda i: (0, 0)),
        ],
        out_specs=[
            pl.BlockSpec((block_rows, TOP_K), lambda i: (i, 0)),
            pl.BlockSpec((block_rows, TOP_K), lambda i: (i, 0)),
        ],
        out_shape=[
            jax.ShapeDtypeStruct((rows, TOP_K), jnp.float32),
            jax.ShapeDtypeStruct((rows, TOP_K), jnp.int32),
        ],
        compiler_params=pltpu.CompilerParams(
            dimension_semantics=("parallel",),
        ),
    )(x_flat, W)


def kernel(x, W):
    batch, seq, hidden = x.shape
    x_flat = x.reshape(batch * seq, hidden)
    top_scores, top_idx = _gate(x_flat, W, 512)
    aux = jnp.zeros((), dtype=x.dtype)
    return (top_scores, top_idx, aux)
